# baseline (speedup 1.0000x reference)
import jax
import jax.numpy as jnp
from jax import lax
from jax.experimental import pallas as pl
from jax.experimental.pallas import tpu as pltpu
from jax.experimental.pallas import tpu_sc as plsc

VOCAB = 49408
MAXPOS = 77
DIM = 768
B = 256
S = 77
LANES = 16
NUM_CORES = 2
NUM_SUBCORES = 16
NW = NUM_CORES * NUM_SUBCORES
SEQ_PER_W = B // NW
RPW = S * SEQ_PER_W
SBLK = 4
BLK = SBLK * SEQ_PER_W
NBLK = -(-S // SBLK)
NBUF = 2


def _body(ids_hbm, token_hbm, pos_hbm, out_hbm, *scratch):
    idx_v, pos_v = scratch[0], scratch[1]
    bufs = scratch[2:2 + NBUF]
    ids_sem, pos_sem = scratch[2 + NBUF], scratch[3 + NBUF]
    gsem = scratch[4 + NBUF:4 + 2 * NBUF]
    osem = scratch[4 + 2 * NBUF:4 + 3 * NBUF]
    wid = lax.axis_index("s") * NUM_CORES + lax.axis_index("c")
    base = wid * RPW
    bcol = pl.multiple_of(wid * SEQ_PER_W, SEQ_PER_W)

    def rows_of(k):
        return min(SBLK, S - k * SBLK) * SEQ_PER_W

    def gather(k):
        n = rows_of(k)
        b = bufs[k % NBUF]
        dst = b if n == BLK else b.at[pl.ds(0, n)]
        return pltpu.async_copy(
            token_hbm.at[idx_v.at[pl.ds(k * BLK, n)]], dst, gsem[k % NBUF]
        )

    def store(k):
        b = bufs[k % NBUF]
        handles = []
        for si in range(rows_of(k) // SEQ_PER_W):
            src = b.at[pl.ds(si * SEQ_PER_W, SEQ_PER_W)]
            dst = out_hbm.at[k * SBLK + si].at[pl.ds(bcol, SEQ_PER_W)]
            handles.append(pltpu.async_copy(src, dst, osem[k % NBUF]))
        return handles

    def add(k):
        b = bufs[k % NBUF]
        for si in range(rows_of(k) // SEQ_PER_W):
            s = k * SBLK + si

            def body(j, carry, si=si, s=s):
                sl = pl.ds(j * LANES, LANES)
                p = pos_v[s, sl]
                for r in range(SEQ_PER_W):
                    row = si * SEQ_PER_W + r
                    b[row, sl] = b[row, sl] + p
                return carry

            lax.fori_loop(0, DIM // LANES, body, 0)

    pltpu.async_copy(ids_hbm.at[pl.ds(base, RPW)], idx_v, ids_sem).wait()
    pos_cp = pltpu.async_copy(pos_hbm, pos_v, pos_sem)

    lookahead = NBUF - 1
    g = {k: gather(k) for k in range(min(lookahead, NBLK))}
    o = {}
    pos_cp.wait()
    for k in range(NBLK):
        g[k].wait()
        add(k)
        o[k] = store(k)
        nk = k + lookahead
        if nk < NBLK:
            if nk >= NBUF:
                for h in o[nk - NBUF]:
                    h.wait()
            g[nk] = gather(nk)
    for k in range(max(0, NBLK - NBUF), NBLK):
        for h in o[k]:
            h.wait()


@jax.jit
def _run(input_ids, token_table, pos_table):
    ids = input_ids.reshape(NW, SEQ_PER_W, S).transpose(0, 2, 1).reshape(-1)
    mesh = plsc.VectorSubcoreMesh(core_axis_name="c", subcore_axis_name="s")
    out = pl.kernel(
        _body,
        out_type=jax.ShapeDtypeStruct((S, B, DIM), jnp.float32),
        mesh=mesh,
        scratch_types=[
            pltpu.VMEM((RPW,), jnp.int32),
            pltpu.VMEM((S, DIM), jnp.float32),
        ] + [pltpu.VMEM((BLK, DIM), jnp.float32)] * NBUF
          + [pltpu.SemaphoreType.DMA] * (2 + 2 * NBUF),
    )(ids, token_table, pos_table)
    return out.transpose(1, 0, 2)


def kernel(ctx_embeddings, input_ids, token_table, pos_table):
    del ctx_embeddings
    return _run(input_ids, token_table, pos_table)


# speedup vs baseline: 1.3123x; 1.3123x over previous
import jax
import jax.numpy as jnp
from jax import lax
from jax.experimental import pallas as pl
from jax.experimental.pallas import tpu as pltpu
from jax.experimental.pallas import tpu_sc as plsc

VOCAB = 49408
MAXPOS = 77
DIM = 768
B = 256
S = 77
LANES = 16
NUM_CORES = 2
NUM_SUBCORES = 16
NW = NUM_CORES * NUM_SUBCORES
SEQ_PER_W = B // NW
RPW = S * SEQ_PER_W
SBLK = 2
BLK = SBLK * SEQ_PER_W
NBLK = -(-S // SBLK)
NBUF = 4


def _body(ids_hbm, token_hbm, pos_hbm, out_hbm, *scratch):
    idx_v, pos_v = scratch[0], scratch[1]
    bufs = scratch[2:2 + NBUF]
    ids_sem, pos_sem = scratch[2 + NBUF], scratch[3 + NBUF]
    gsem = scratch[4 + NBUF:4 + 2 * NBUF]
    osem = scratch[4 + 2 * NBUF:4 + 3 * NBUF]
    wid = lax.axis_index("s") * NUM_CORES + lax.axis_index("c")
    base = wid * RPW
    bcol = pl.multiple_of(wid * SEQ_PER_W, SEQ_PER_W)

    def rows_of(k):
        return min(SBLK, S - k * SBLK) * SEQ_PER_W

    def gather(k):
        n = rows_of(k)
        b = bufs[k % NBUF]
        dst = b if n == BLK else b.at[pl.ds(0, n)]
        return pltpu.async_copy(
            token_hbm.at[idx_v.at[pl.ds(k * BLK, n)]], dst, gsem[k % NBUF]
        )

    def store(k):
        b = bufs[k % NBUF]
        handles = []
        for si in range(rows_of(k) // SEQ_PER_W):
            src = b.at[pl.ds(si * SEQ_PER_W, SEQ_PER_W)]
            dst = out_hbm.at[k * SBLK + si].at[pl.ds(bcol, SEQ_PER_W)]
            handles.append(pltpu.async_copy(src, dst, osem[k % NBUF]))
        return handles

    def add(k):
        b = bufs[k % NBUF]
        for si in range(rows_of(k) // SEQ_PER_W):
            s = k * SBLK + si

            def body(j, carry, si=si, s=s):
                sl = pl.ds(j * LANES, LANES)
                p = pos_v[s, sl]
                for r in range(SEQ_PER_W):
                    row = si * SEQ_PER_W + r
                    b[row, sl] = b[row, sl] + p
                return carry

            lax.fori_loop(0, DIM // LANES, body, 0)

    pltpu.async_copy(ids_hbm.at[pl.ds(base, RPW)], idx_v, ids_sem).wait()
    pos_cp = pltpu.async_copy(pos_hbm, pos_v, pos_sem)

    lookahead = NBUF - 1
    g = {k: gather(k) for k in range(min(lookahead, NBLK))}
    o = {}
    pos_cp.wait()
    for k in range(NBLK):
        g[k].wait()
        add(k)
        o[k] = store(k)
        nk = k + lookahead
        if nk < NBLK:
            if nk >= NBUF:
                for h in o[nk - NBUF]:
                    h.wait()
            g[nk] = gather(nk)
    for k in range(max(0, NBLK - NBUF), NBLK):
        for h in o[k]:
            h.wait()


@jax.jit
def _run(input_ids, token_table, pos_table):
    ids = input_ids.reshape(NW, SEQ_PER_W, S).transpose(0, 2, 1).reshape(-1)
    mesh = plsc.VectorSubcoreMesh(core_axis_name="c", subcore_axis_name="s")
    out = pl.kernel(
        _body,
        out_type=jax.ShapeDtypeStruct((S, B, DIM), jnp.float32),
        mesh=mesh,
        scratch_types=[
            pltpu.VMEM((RPW,), jnp.int32),
            pltpu.VMEM((S, DIM), jnp.float32),
        ] + [pltpu.VMEM((BLK, DIM), jnp.float32)] * NBUF
          + [pltpu.SemaphoreType.DMA] * (2 + 2 * NBUF),
    )(ids, token_table, pos_table)
    return out.transpose(1, 0, 2)


def kernel(ctx_embeddings, input_ids, token_table, pos_table):
    del ctx_embeddings
    return _run(input_ids, token_table, pos_table)


# speedup vs baseline: 1.3295x; 1.0131x over previous
import jax
import jax.numpy as jnp
from jax import lax
from jax.experimental import pallas as pl
from jax.experimental.pallas import tpu as pltpu
from jax.experimental.pallas import tpu_sc as plsc

VOCAB = 49408
MAXPOS = 77
DIM = 768
B = 256
S = 77
LANES = 16
NUM_CORES = 2
NUM_SUBCORES = 16
NW = NUM_CORES * NUM_SUBCORES
SEQ_PER_W = B // NW
RPW = S * SEQ_PER_W
SBLK = 2
BLK = SBLK * SEQ_PER_W
NBLK = -(-S // SBLK)
NBUF = 5


def _body(ids_hbm, token_hbm, pos_hbm, out_hbm, *scratch):
    idx_v, pos_v = scratch[0], scratch[1]
    bufs = scratch[2:2 + NBUF]
    ids_sem, pos_sem = scratch[2 + NBUF], scratch[3 + NBUF]
    gsem = scratch[4 + NBUF:4 + 2 * NBUF]
    osem = scratch[4 + 2 * NBUF:4 + 3 * NBUF]
    wid = lax.axis_index("s") * NUM_CORES + lax.axis_index("c")
    base = wid * RPW
    bcol = pl.multiple_of(wid * SEQ_PER_W, SEQ_PER_W)

    def rows_of(k):
        return min(SBLK, S - k * SBLK) * SEQ_PER_W

    def gather(k):
        n = rows_of(k)
        b = bufs[k % NBUF]
        dst = b if n == BLK else b.at[pl.ds(0, n)]
        return pltpu.async_copy(
            token_hbm.at[idx_v.at[pl.ds(k * BLK, n)]], dst, gsem[k % NBUF]
        )

    def store(k):
        b = bufs[k % NBUF]
        handles = []
        for si in range(rows_of(k) // SEQ_PER_W):
            src = b.at[pl.ds(si * SEQ_PER_W, SEQ_PER_W)]
            dst = out_hbm.at[k * SBLK + si].at[pl.ds(bcol, SEQ_PER_W)]
            handles.append(pltpu.async_copy(src, dst, osem[k % NBUF]))
        return handles

    def add(k):
        b = bufs[k % NBUF]
        for si in range(rows_of(k) // SEQ_PER_W):
            s = k * SBLK + si

            def body(j, carry, si=si, s=s):
                sl = pl.ds(j * LANES, LANES)
                p = pos_v[s, sl]
                for r in range(SEQ_PER_W):
                    row = si * SEQ_PER_W + r
                    b[row, sl] = b[row, sl] + p
                return carry

            lax.fori_loop(0, DIM // LANES, body, 0)

    pltpu.async_copy(ids_hbm.at[pl.ds(base, RPW)], idx_v, ids_sem).wait()
    pos_cp = pltpu.async_copy(pos_hbm, pos_v, pos_sem)

    lookahead = NBUF - 1
    g = {k: gather(k) for k in range(min(lookahead, NBLK))}
    o = {}
    pos_cp.wait()
    for k in range(NBLK):
        g[k].wait()
        add(k)
        o[k] = store(k)
        nk = k + lookahead
        if nk < NBLK:
            if nk >= NBUF:
                for h in o[nk - NBUF]:
                    h.wait()
            g[nk] = gather(nk)
    for k in range(max(0, NBLK - NBUF), NBLK):
        for h in o[k]:
            h.wait()


@jax.jit
def _run(input_ids, token_table, pos_table):
    ids = input_ids.reshape(NW, SEQ_PER_W, S).transpose(0, 2, 1).reshape(-1)
    mesh = plsc.VectorSubcoreMesh(core_axis_name="c", subcore_axis_name="s")
    out = pl.kernel(
        _body,
        out_type=jax.ShapeDtypeStruct((S, B, DIM), jnp.float32),
        mesh=mesh,
        scratch_types=[
            pltpu.VMEM((RPW,), jnp.int32),
            pltpu.VMEM((S, DIM), jnp.float32),
        ] + [pltpu.VMEM((BLK, DIM), jnp.float32)] * NBUF
          + [pltpu.SemaphoreType.DMA] * (2 + 2 * NBUF),
    )(ids, token_table, pos_table)
    return out.transpose(1, 0, 2)


def kernel(ctx_embeddings, input_ids, token_table, pos_table):
    del ctx_embeddings
    return _run(input_ids, token_table, pos_table)
